# baseline (speedup 1.0000x reference)
import functools

import jax
import jax.numpy as jnp
from jax import lax
from jax.experimental import pallas as pl
from jax.experimental.pallas import tpu as pltpu
from jax.experimental.pallas import tpu_sc as plsc

N = 10000
E = 320000
NPAD = 10240
NC = 2
NS = 16
NW = NC * NS
EPT = E // NW
CHUNK = 80
NCHUNK = EPT // CHUNK
NBUF = 5
LEAD = 2
NROW = 10240
RPT = NROW // NS
COLS = NPAD // NS
D = 64

_mesh = plsc.VectorSubcoreMesh(core_axis_name="c", subcore_axis_name="s")
_sc_params = pltpu.CompilerParams(
    needs_layout_passes=False, use_tc_tiling_on_sc=False,
    skip_device_barrier=True)


def _deg_body(dst_hbm, out_hbm, dstv, hist, shared, redbuf):
    c = lax.axis_index("c")
    s = lax.axis_index("s")
    wid = c * NS + s
    zeros16 = jnp.zeros((16,), jnp.float32)
    ones16 = jnp.ones((16,), jnp.float32)

    def zero(i, _):
        hist[pl.ds(i * 16, 16)] = zeros16
        return ()
    lax.fori_loop(0, NPAD // 16, zero, ())

    pltpu.sync_copy(dst_hbm.at[wid], dstv)

    def count(j, _):
        idx = dstv[pl.ds(j * 16, 16)]
        plsc.addupdate_scatter(hist, [idx], ones16)
        return ()
    lax.fori_loop(0, EPT // 16, count, ())

    pltpu.sync_copy(hist, shared.at[s])
    plsc.subcore_barrier()

    pltpu.sync_copy(shared.at[:, pl.ds(s * COLS, COLS)], redbuf)

    def red(jc, _):
        acc = jnp.zeros((16,), jnp.float32)
        for r in range(NS):
            acc = acc + redbuf[r, pl.ds(jc * 16, 16)]
        hist[pl.ds(jc * 16, 16)] = acc
        return ()
    lax.fori_loop(0, COLS // 16, red, ())
    pltpu.sync_copy(hist.at[pl.ds(0, COLS)], out_hbm.at[c, pl.ds(s * COLS, COLS)])


_deg_kernel = functools.partial(
    pl.kernel,
    out_type=jax.ShapeDtypeStruct((NC, NPAD), jnp.float32),
    mesh=_mesh,
    scratch_types=[
        pltpu.VMEM((EPT,), jnp.int32),
        pltpu.VMEM((NPAD,), jnp.float32),
        pltpu.VMEM_SHARED((NS, NPAD), jnp.float32),
        pltpu.VMEM((NS, COLS), jnp.float32),
    ],
    compiler_params=_sc_params,
)(_deg_body)


def _agg_body(table, srcr, dstr, out_hbm, srcv, dstv, acc, rows, gsems, ssems):
    c = lax.axis_index("c")
    s = lax.axis_index("s")
    wid = c * NS + s
    zeros16 = jnp.zeros((16,), jnp.float32)

    def zrow(r, _):
        for k in range(D // 16):
            rows[0][r, pl.ds(k * 16, 16)] = zeros16
        return ()
    lax.fori_loop(0, CHUNK, zrow, ())
    for b in range(RPT // CHUNK):
        pltpu.sync_copy(rows[0], acc.at[pl.ds(s * RPT + b * CHUNK, CHUNK)])
    plsc.subcore_barrier()

    pltpu.sync_copy(srcr.at[wid], srcv)
    pltpu.sync_copy(dstr.at[wid], dstv)

    for b in range(LEAD):
        pltpu.async_copy(table.at[srcv.at[b]], rows[b], gsems[b])

    def outer(g, _):
        for b in range(NBUF):
            jj = g * NBUF + b
            bg = (b + LEAD) % NBUF
            pre = jj + LEAD

            @pl.when(jnp.logical_and(pre < NCHUNK, pre >= NBUF))
            def _():
                pltpu.make_async_copy(
                    rows[bg], acc.at[dstv.at[pre - NBUF]], ssems[bg]).wait()

            @pl.when(pre < NCHUNK)
            def _():
                pltpu.async_copy(table.at[srcv.at[pre]], rows[bg], gsems[bg])

            pltpu.make_async_copy(table.at[srcv.at[jj]], rows[b], gsems[b]).wait()
            pltpu.async_copy(rows[b], acc.at[dstv.at[jj]], ssems[b], add=True)
        return ()
    lax.fori_loop(0, NCHUNK // NBUF, outer, ())

    for b in range(NBUF):
        jj = NCHUNK - NBUF + b
        pltpu.make_async_copy(rows[b], acc.at[dstv.at[jj]], ssems[b]).wait()

    plsc.subcore_barrier()
    pltpu.sync_copy(acc.at[pl.ds(s * RPT, RPT)],
                    out_hbm.at[c, pl.ds(s * RPT, RPT)])


_agg = functools.partial(
    pl.kernel,
    out_type=jax.ShapeDtypeStruct((NC, NROW, D), jnp.float32),
    mesh=_mesh,
    scratch_types=[
        pltpu.VMEM((NCHUNK, CHUNK), jnp.int32),
        pltpu.VMEM((NCHUNK, CHUNK), jnp.int32),
        pltpu.VMEM_SHARED((NROW, D), jnp.float32),
        [pltpu.VMEM((CHUNK, D), jnp.float32) for _ in range(NBUF)],
        [pltpu.SemaphoreType.DMA for _ in range(NBUF)],
        [pltpu.SemaphoreType.DMA for _ in range(NBUF)],
    ],
    compiler_params=_sc_params,
)(_agg_body)


def _agg2_body(table_a, table_b, srcr, dstr, out_hbm,
               srcv, dstv, acc, rows, gsems, ssems):
    c = lax.axis_index("c")
    s = lax.axis_index("s")
    zeros16 = jnp.zeros((16,), jnp.float32)

    def zrow(r, _):
        for k in range(D // 16):
            rows[0][r, pl.ds(k * 16, 16)] = zeros16
        return ()
    lax.fori_loop(0, CHUNK, zrow, ())
    for b in range(RPT // CHUNK):
        pltpu.sync_copy(rows[0], acc.at[pl.ds(s * RPT + b * CHUNK, CHUNK)])
    plsc.subcore_barrier()

    def gissue(idx, b):
        @pl.when(c == 0)
        def _():
            pltpu.async_copy(table_a.at[idx], rows[b], gsems[b])

        @pl.when(c == 1)
        def _():
            pltpu.async_copy(table_b.at[idx], rows[b], gsems[b])

    def phase(p, _):
        pltpu.sync_copy(srcr.at[s, p], srcv)
        pltpu.sync_copy(dstr.at[s, p], dstv)

        for b in range(LEAD):
            gissue(srcv.at[b], b)

        def outer(g, _):
            for b in range(NBUF):
                jj = g * NBUF + b
                bg = (b + LEAD) % NBUF
                pre = jj + LEAD

                @pl.when(jnp.logical_and(pre < NCHUNK, pre >= NBUF))
                def _():
                    pltpu.make_async_copy(
                        rows[bg], acc.at[dstv.at[pre - NBUF]], ssems[bg]).wait()

                @pl.when(pre < NCHUNK)
                def _():
                    gissue(srcv.at[pre], bg)

                pltpu.make_async_copy(
                    table_a.at[srcv.at[jj]], rows[b], gsems[b]).wait()
                pltpu.async_copy(rows[b], acc.at[dstv.at[jj]], ssems[b], add=True)
            return ()
        lax.fori_loop(0, NCHUNK // NBUF, outer, ())

        for b in range(NBUF):
            jj = NCHUNK - NBUF + b
            pltpu.make_async_copy(rows[b], acc.at[dstv.at[jj]], ssems[b]).wait()
        return ()
    lax.fori_loop(0, 2, phase, ())

    plsc.subcore_barrier()
    pltpu.sync_copy(acc.at[pl.ds(s * RPT, RPT)],
                    out_hbm.at[c, pl.ds(s * RPT, RPT)])


_agg2 = functools.partial(
    pl.kernel,
    out_type=jax.ShapeDtypeStruct((NC, NROW, D), jnp.float32),
    mesh=_mesh,
    scratch_types=[
        pltpu.VMEM((NCHUNK, CHUNK), jnp.int32),
        pltpu.VMEM((NCHUNK, CHUNK), jnp.int32),
        pltpu.VMEM_SHARED((NROW, D), jnp.float32),
        [pltpu.VMEM((CHUNK, D), jnp.float32) for _ in range(NBUF)],
        [pltpu.SemaphoreType.DMA for _ in range(NBUF)],
        [pltpu.SemaphoreType.DMA for _ in range(NBUF)],
    ],
    compiler_params=_sc_params,
)(_agg2_body)


_BLK = 10000
_GRID = N // _BLK


def _row_spec(d):
    return pl.BlockSpec((_BLK, d), lambda i: (i, 0))


def _full_spec(a, b):
    return pl.BlockSpec((a, b), lambda i: (0, 0))


def _tc_call(body, in_specs, out_specs, out_shape):
    return pl.pallas_call(
        body, grid=(_GRID,), in_specs=in_specs,
        out_specs=out_specs, out_shape=out_shape)


def _tcA_body(c0, c1, x, w1, dinv_o, h1a_o, h1b_o):
    dinv = lax.rsqrt(c0[...] + c1[...] + 1.0)
    dinv_o[...] = dinv
    h1p = dinv * jnp.dot(x[...], w1[...], preferred_element_type=jnp.float32)
    h1a_o[...] = h1p[:, :D]
    h1b_o[...] = h1p[:, D:]


def _tcB_body(ra, rb, h1a, h1b, dinv, b1, w2, h2p_o):
    lo = dinv[...] * (ra[...] + h1a[...]) + b1[...][:, :D]
    hi = dinv[...] * (rb[...] + h1b[...]) + b1[...][:, D:]
    x2 = jnp.maximum(jnp.concatenate([lo, hi], axis=1), 0.0)
    h2p_o[...] = dinv[...] * jnp.dot(x2, w2[...], preferred_element_type=jnp.float32)


def _tcC_body(r0, r1, h2p, dinv, b2, z_o, zp_o):
    z = dinv[...] * (r0[...] + r1[...] + h2p[...]) + b2[...]
    z_o[...] = z
    zp_o[...] = dinv[...] * z


def _tcD_body(r0, r1, zp, dinv, w3, b3, w4, h4a_o, h4b_o):
    agg = dinv[...] * (r0[...] + r1[...] + zp[...])
    g = jnp.maximum(jnp.dot(agg, w3[...], preferred_element_type=jnp.float32)
                    + b3[...], 0.0)
    h4p = dinv[...] * jnp.dot(g, w4[...], preferred_element_type=jnp.float32)
    h4a_o[...] = h4p[:, :D]
    h4b_o[...] = h4p[:, D:]


def _tcE_body(ra, rb, h4a, h4b, dinv, b4, xh_o):
    lo = dinv[...] * (ra[...] + h4a[...]) + b4[...][:, :D]
    hi = dinv[...] * (rb[...] + h4b[...]) + b4[...][:, D:]
    xh_o[...] = jnp.concatenate([lo, hi], axis=1)


def kernel(x, edge_index, W1, b1, W2, b2, W3, b3, W4, b4):
    src = edge_index[0].reshape(NW, NCHUNK, CHUNK)
    dst = edge_index[1].reshape(NW, NCHUNK, CHUNK)
    src2 = edge_index[0].reshape(NS, 2, NCHUNK, CHUNK)
    dst2 = edge_index[1].reshape(NS, 2, NCHUNK, CHUNK)
    dst_flat = edge_index[1].reshape(NW, EPT)

    cnt = _deg_kernel(dst_flat)
    c0 = cnt[0].reshape(NPAD, 1)
    c1 = cnt[1].reshape(NPAD, 1)

    dinv, h1a, h1b = _tc_call(
        _tcA_body,
        [_row_spec(1), _row_spec(1), _row_spec(128), _full_spec(128, 128)],
        [_row_spec(1), _row_spec(D), _row_spec(D)],
        [jax.ShapeDtypeStruct((N, 1), jnp.float32),
         jax.ShapeDtypeStruct((N, D), jnp.float32),
         jax.ShapeDtypeStruct((N, D), jnp.float32)],
    )(c0, c1, x, W1)

    r1 = _agg2(h1a, h1b, src2, dst2)
    h2p = _tc_call(
        _tcB_body,
        [_row_spec(D)] * 2 + [_row_spec(D), _row_spec(D), _row_spec(1),
                              _full_spec(1, 128), _full_spec(128, 64)],
        [_row_spec(D)],
        [jax.ShapeDtypeStruct((N, D), jnp.float32)],
    )(r1[0], r1[1], h1a, h1b, dinv, b1.reshape(1, 128), W2)[0]

    r2 = _agg(h2p, src, dst)
    z, zp = _tc_call(
        _tcC_body,
        [_row_spec(D), _row_spec(D), _row_spec(D), _row_spec(1),
         _full_spec(1, 64)],
        [_row_spec(D), _row_spec(D)],
        [jax.ShapeDtypeStruct((N, D), jnp.float32),
         jax.ShapeDtypeStruct((N, D), jnp.float32)],
    )(r2[0], r2[1], h2p, dinv, b2.reshape(1, 64))

    r3 = _agg(zp, src, dst)
    h4a, h4b = _tc_call(
        _tcD_body,
        [_row_spec(D), _row_spec(D), _row_spec(D), _row_spec(1),
         _full_spec(64, 128), _full_spec(1, 128), _full_spec(128, 128)],
        [_row_spec(D), _row_spec(D)],
        [jax.ShapeDtypeStruct((N, D), jnp.float32),
         jax.ShapeDtypeStruct((N, D), jnp.float32)],
    )(r3[0], r3[1], zp, dinv, W3, b3.reshape(1, 128), W4)

    r4 = _agg2(h4a, h4b, src2, dst2)
    x_hat = _tc_call(
        _tcE_body,
        [_row_spec(D)] * 2 + [_row_spec(D), _row_spec(D), _row_spec(1),
                              _full_spec(1, 128)],
        [_row_spec(128)],
        [jax.ShapeDtypeStruct((N, 128), jnp.float32)],
    )(r4[0], r4[1], h4a, h4b, dinv, b4.reshape(1, 128))[0]

    return (x_hat, z)


# speedup vs baseline: 1.0622x; 1.0622x over previous
import functools

import jax
import jax.numpy as jnp
from jax import lax
from jax.experimental import pallas as pl
from jax.experimental.pallas import tpu as pltpu
from jax.experimental.pallas import tpu_sc as plsc

N = 10000
E = 320000
NPAD = 10240
NC = 2
NS = 16
NW = NC * NS
EPT = E // NW
CHUNK = 80
NCHUNK = EPT // CHUNK
NBUF = 5
LEAD = 3
NROW = 10240
RPT = NROW // NS
COLS = NPAD // NS
D = 64

_mesh = plsc.VectorSubcoreMesh(core_axis_name="c", subcore_axis_name="s")
_sc_params = pltpu.CompilerParams(
    needs_layout_passes=False, use_tc_tiling_on_sc=False,
    skip_device_barrier=True)


def _deg_body(dst_hbm, out_hbm, dstv, hist, shared, redbuf):
    c = lax.axis_index("c")
    s = lax.axis_index("s")
    wid = c * NS + s
    zeros16 = jnp.zeros((16,), jnp.float32)
    ones16 = jnp.ones((16,), jnp.float32)

    def zero(i, _):
        hist[pl.ds(i * 16, 16)] = zeros16
        return ()
    lax.fori_loop(0, NPAD // 16, zero, ())

    pltpu.sync_copy(dst_hbm.at[wid], dstv)

    def count(j, _):
        idx = dstv[pl.ds(j * 16, 16)]
        plsc.addupdate_scatter(hist, [idx], ones16)
        return ()
    lax.fori_loop(0, EPT // 16, count, ())

    pltpu.sync_copy(hist, shared.at[s])
    plsc.subcore_barrier()

    pltpu.sync_copy(shared.at[:, pl.ds(s * COLS, COLS)], redbuf)

    def red(jc, _):
        acc = jnp.zeros((16,), jnp.float32)
        for r in range(NS):
            acc = acc + redbuf[r, pl.ds(jc * 16, 16)]
        hist[pl.ds(jc * 16, 16)] = acc
        return ()
    lax.fori_loop(0, COLS // 16, red, ())
    pltpu.sync_copy(hist.at[pl.ds(0, COLS)], out_hbm.at[c, pl.ds(s * COLS, COLS)])


_deg_kernel = functools.partial(
    pl.kernel,
    out_type=jax.ShapeDtypeStruct((NC, NPAD), jnp.float32),
    mesh=_mesh,
    scratch_types=[
        pltpu.VMEM((EPT,), jnp.int32),
        pltpu.VMEM((NPAD,), jnp.float32),
        pltpu.VMEM_SHARED((NS, NPAD), jnp.float32),
        pltpu.VMEM((NS, COLS), jnp.float32),
    ],
    compiler_params=_sc_params,
)(_deg_body)


def _agg_body(table, srcr, dstr, out_hbm, srcv, dstv, acc, rows, gsems, ssems):
    c = lax.axis_index("c")
    s = lax.axis_index("s")
    wid = c * NS + s
    zeros16 = jnp.zeros((16,), jnp.float32)

    def zrow(r, _):
        for k in range(D // 16):
            rows[0][r, pl.ds(k * 16, 16)] = zeros16
        return ()
    lax.fori_loop(0, CHUNK, zrow, ())
    for b in range(RPT // CHUNK):
        pltpu.sync_copy(rows[0], acc.at[pl.ds(s * RPT + b * CHUNK, CHUNK)])
    plsc.subcore_barrier()

    pltpu.sync_copy(srcr.at[wid], srcv)
    pltpu.sync_copy(dstr.at[wid], dstv)

    for b in range(LEAD):
        pltpu.async_copy(table.at[srcv.at[b]], rows[b], gsems[b])

    def outer(g, _):
        for b in range(NBUF):
            jj = g * NBUF + b
            bg = (b + LEAD) % NBUF
            pre = jj + LEAD

            @pl.when(jnp.logical_and(pre < NCHUNK, pre >= NBUF))
            def _():
                pltpu.make_async_copy(
                    rows[bg], acc.at[dstv.at[pre - NBUF]], ssems[bg]).wait()

            @pl.when(pre < NCHUNK)
            def _():
                pltpu.async_copy(table.at[srcv.at[pre]], rows[bg], gsems[bg])

            pltpu.make_async_copy(table.at[srcv.at[jj]], rows[b], gsems[b]).wait()
            pltpu.async_copy(rows[b], acc.at[dstv.at[jj]], ssems[b], add=True)
        return ()
    lax.fori_loop(0, NCHUNK // NBUF, outer, ())

    for b in range(NBUF):
        jj = NCHUNK - NBUF + b
        pltpu.make_async_copy(rows[b], acc.at[dstv.at[jj]], ssems[b]).wait()

    plsc.subcore_barrier()
    pltpu.sync_copy(acc.at[pl.ds(s * RPT, RPT)],
                    out_hbm.at[c, pl.ds(s * RPT, RPT)])


_agg = functools.partial(
    pl.kernel,
    out_type=jax.ShapeDtypeStruct((NC, NROW, D), jnp.float32),
    mesh=_mesh,
    scratch_types=[
        pltpu.VMEM((NCHUNK, CHUNK), jnp.int32),
        pltpu.VMEM((NCHUNK, CHUNK), jnp.int32),
        pltpu.VMEM_SHARED((NROW, D), jnp.float32),
        [pltpu.VMEM((CHUNK, D), jnp.float32) for _ in range(NBUF)],
        [pltpu.SemaphoreType.DMA for _ in range(NBUF)],
        [pltpu.SemaphoreType.DMA for _ in range(NBUF)],
    ],
    compiler_params=_sc_params,
)(_agg_body)


def _agg2_body(table_a, table_b, srcr, dstr, out_hbm,
               srcv, dstv, acc, rows, gsems, ssems):
    c = lax.axis_index("c")
    s = lax.axis_index("s")
    zeros16 = jnp.zeros((16,), jnp.float32)

    def zrow(r, _):
        for k in range(D // 16):
            rows[0][r, pl.ds(k * 16, 16)] = zeros16
        return ()
    lax.fori_loop(0, CHUNK, zrow, ())
    for b in range(RPT // CHUNK):
        pltpu.sync_copy(rows[0], acc.at[pl.ds(s * RPT + b * CHUNK, CHUNK)])
    plsc.subcore_barrier()

    def gissue(idx, b):
        @pl.when(c == 0)
        def _():
            pltpu.async_copy(table_a.at[idx], rows[b], gsems[b])

        @pl.when(c == 1)
        def _():
            pltpu.async_copy(table_b.at[idx], rows[b], gsems[b])

    def phase(p, _):
        pltpu.sync_copy(srcr.at[s, p], srcv)
        pltpu.sync_copy(dstr.at[s, p], dstv)

        for b in range(LEAD):
            gissue(srcv.at[b], b)

        def outer(g, _):
            for b in range(NBUF):
                jj = g * NBUF + b
                bg = (b + LEAD) % NBUF
                pre = jj + LEAD

                @pl.when(jnp.logical_and(pre < NCHUNK, pre >= NBUF))
                def _():
                    pltpu.make_async_copy(
                        rows[bg], acc.at[dstv.at[pre - NBUF]], ssems[bg]).wait()

                @pl.when(pre < NCHUNK)
                def _():
                    gissue(srcv.at[pre], bg)

                pltpu.make_async_copy(
                    table_a.at[srcv.at[jj]], rows[b], gsems[b]).wait()
                pltpu.async_copy(rows[b], acc.at[dstv.at[jj]], ssems[b], add=True)
            return ()
        lax.fori_loop(0, NCHUNK // NBUF, outer, ())

        for b in range(NBUF):
            jj = NCHUNK - NBUF + b
            pltpu.make_async_copy(rows[b], acc.at[dstv.at[jj]], ssems[b]).wait()
        return ()
    lax.fori_loop(0, 2, phase, ())

    plsc.subcore_barrier()
    pltpu.sync_copy(acc.at[pl.ds(s * RPT, RPT)],
                    out_hbm.at[c, pl.ds(s * RPT, RPT)])


_agg2 = functools.partial(
    pl.kernel,
    out_type=jax.ShapeDtypeStruct((NC, NROW, D), jnp.float32),
    mesh=_mesh,
    scratch_types=[
        pltpu.VMEM((NCHUNK, CHUNK), jnp.int32),
        pltpu.VMEM((NCHUNK, CHUNK), jnp.int32),
        pltpu.VMEM_SHARED((NROW, D), jnp.float32),
        [pltpu.VMEM((CHUNK, D), jnp.float32) for _ in range(NBUF)],
        [pltpu.SemaphoreType.DMA for _ in range(NBUF)],
        [pltpu.SemaphoreType.DMA for _ in range(NBUF)],
    ],
    compiler_params=_sc_params,
)(_agg2_body)


_BLK = 2000
_GRID = N // _BLK


def _row_spec(d):
    return pl.BlockSpec((_BLK, d), lambda i: (i, 0))


def _full_spec(a, b):
    return pl.BlockSpec((a, b), lambda i: (0, 0))


def _tc_call(body, in_specs, out_specs, out_shape):
    return pl.pallas_call(
        body, grid=(_GRID,), in_specs=in_specs,
        out_specs=out_specs, out_shape=out_shape)


def _tcA_body(c0, c1, x, w1, dinv_o, h1a_o, h1b_o):
    dinv = lax.rsqrt(c0[...] + c1[...] + 1.0)
    dinv_o[...] = dinv
    h1p = dinv * jnp.dot(x[...], w1[...], preferred_element_type=jnp.float32)
    h1a_o[...] = h1p[:, :D]
    h1b_o[...] = h1p[:, D:]


def _tcB_body(ra, rb, h1a, h1b, dinv, b1, w2, h2p_o):
    lo = dinv[...] * (ra[...] + h1a[...]) + b1[...][:, :D]
    hi = dinv[...] * (rb[...] + h1b[...]) + b1[...][:, D:]
    x2 = jnp.maximum(jnp.concatenate([lo, hi], axis=1), 0.0)
    h2p_o[...] = dinv[...] * jnp.dot(x2, w2[...], preferred_element_type=jnp.float32)


def _tcC_body(r0, r1, h2p, dinv, b2, z_o, zp_o):
    z = dinv[...] * (r0[...] + r1[...] + h2p[...]) + b2[...]
    z_o[...] = z
    zp_o[...] = dinv[...] * z


def _tcD_body(r0, r1, zp, dinv, w3, b3, w4, h4a_o, h4b_o):
    agg = dinv[...] * (r0[...] + r1[...] + zp[...])
    g = jnp.maximum(jnp.dot(agg, w3[...], preferred_element_type=jnp.float32)
                    + b3[...], 0.0)
    h4p = dinv[...] * jnp.dot(g, w4[...], preferred_element_type=jnp.float32)
    h4a_o[...] = h4p[:, :D]
    h4b_o[...] = h4p[:, D:]


def _tcE_body(ra, rb, h4a, h4b, dinv, b4, xh_o):
    lo = dinv[...] * (ra[...] + h4a[...]) + b4[...][:, :D]
    hi = dinv[...] * (rb[...] + h4b[...]) + b4[...][:, D:]
    xh_o[...] = jnp.concatenate([lo, hi], axis=1)


def kernel(x, edge_index, W1, b1, W2, b2, W3, b3, W4, b4):
    src = edge_index[0].reshape(NW, NCHUNK, CHUNK)
    dst = edge_index[1].reshape(NW, NCHUNK, CHUNK)
    src2 = edge_index[0].reshape(NS, 2, NCHUNK, CHUNK)
    dst2 = edge_index[1].reshape(NS, 2, NCHUNK, CHUNK)
    dst_flat = edge_index[1].reshape(NW, EPT)

    cnt = _deg_kernel(dst_flat)
    c0 = cnt[0].reshape(NPAD, 1)
    c1 = cnt[1].reshape(NPAD, 1)

    dinv, h1a, h1b = _tc_call(
        _tcA_body,
        [_row_spec(1), _row_spec(1), _row_spec(128), _full_spec(128, 128)],
        [_row_spec(1), _row_spec(D), _row_spec(D)],
        [jax.ShapeDtypeStruct((N, 1), jnp.float32),
         jax.ShapeDtypeStruct((N, D), jnp.float32),
         jax.ShapeDtypeStruct((N, D), jnp.float32)],
    )(c0, c1, x, W1)

    r1 = _agg2(h1a, h1b, src2, dst2)
    h2p = _tc_call(
        _tcB_body,
        [_row_spec(D)] * 2 + [_row_spec(D), _row_spec(D), _row_spec(1),
                              _full_spec(1, 128), _full_spec(128, 64)],
        [_row_spec(D)],
        [jax.ShapeDtypeStruct((N, D), jnp.float32)],
    )(r1[0], r1[1], h1a, h1b, dinv, b1.reshape(1, 128), W2)[0]

    r2 = _agg(h2p, src, dst)
    z, zp = _tc_call(
        _tcC_body,
        [_row_spec(D), _row_spec(D), _row_spec(D), _row_spec(1),
         _full_spec(1, 64)],
        [_row_spec(D), _row_spec(D)],
        [jax.ShapeDtypeStruct((N, D), jnp.float32),
         jax.ShapeDtypeStruct((N, D), jnp.float32)],
    )(r2[0], r2[1], h2p, dinv, b2.reshape(1, 64))

    r3 = _agg(zp, src, dst)
    h4a, h4b = _tc_call(
        _tcD_body,
        [_row_spec(D), _row_spec(D), _row_spec(D), _row_spec(1),
         _full_spec(64, 128), _full_spec(1, 128), _full_spec(128, 128)],
        [_row_spec(D), _row_spec(D)],
        [jax.ShapeDtypeStruct((N, D), jnp.float32),
         jax.ShapeDtypeStruct((N, D), jnp.float32)],
    )(r3[0], r3[1], zp, dinv, W3, b3.reshape(1, 128), W4)

    r4 = _agg2(h4a, h4b, src2, dst2)
    x_hat = _tc_call(
        _tcE_body,
        [_row_spec(D)] * 2 + [_row_spec(D), _row_spec(D), _row_spec(1),
                              _full_spec(1, 128)],
        [_row_spec(128)],
        [jax.ShapeDtypeStruct((N, 128), jnp.float32)],
    )(r4[0], r4[1], h4a, h4b, dinv, b4.reshape(1, 128))[0]

    return (x_hat, z)


# speedup vs baseline: 1.0824x; 1.0190x over previous
import functools

import jax
import jax.numpy as jnp
from jax import lax
from jax.experimental import pallas as pl
from jax.experimental.pallas import tpu as pltpu
from jax.experimental.pallas import tpu_sc as plsc

N = 10000
E = 320000
NPAD = 10240
NC = 2
NS = 16
NW = NC * NS
EPT = E // NW
CHUNK = 80
NCHUNK = EPT // CHUNK
NBUF = 5
LEAD = 4
NROW = 10240
RPT = NROW // NS
COLS = NPAD // NS
D = 64

_mesh = plsc.VectorSubcoreMesh(core_axis_name="c", subcore_axis_name="s")
_sc_params = pltpu.CompilerParams(
    needs_layout_passes=False, use_tc_tiling_on_sc=False,
    skip_device_barrier=True)


def _deg_body(dst_hbm, out_hbm, dstv, hist, shared, redbuf):
    c = lax.axis_index("c")
    s = lax.axis_index("s")
    wid = c * NS + s
    zeros16 = jnp.zeros((16,), jnp.float32)
    ones16 = jnp.ones((16,), jnp.float32)

    def zero(i, _):
        hist[pl.ds(i * 16, 16)] = zeros16
        return ()
    lax.fori_loop(0, NPAD // 16, zero, ())

    pltpu.sync_copy(dst_hbm.at[wid], dstv)

    def count(j, _):
        idx = dstv[pl.ds(j * 16, 16)]
        plsc.addupdate_scatter(hist, [idx], ones16)
        return ()
    lax.fori_loop(0, EPT // 16, count, ())

    pltpu.sync_copy(hist, shared.at[s])
    plsc.subcore_barrier()

    pltpu.sync_copy(shared.at[:, pl.ds(s * COLS, COLS)], redbuf)

    def red(jc, _):
        acc = jnp.zeros((16,), jnp.float32)
        for r in range(NS):
            acc = acc + redbuf[r, pl.ds(jc * 16, 16)]
        hist[pl.ds(jc * 16, 16)] = acc
        return ()
    lax.fori_loop(0, COLS // 16, red, ())
    pltpu.sync_copy(hist.at[pl.ds(0, COLS)], out_hbm.at[c, pl.ds(s * COLS, COLS)])


_deg_kernel = functools.partial(
    pl.kernel,
    out_type=jax.ShapeDtypeStruct((NC, NPAD), jnp.float32),
    mesh=_mesh,
    scratch_types=[
        pltpu.VMEM((EPT,), jnp.int32),
        pltpu.VMEM((NPAD,), jnp.float32),
        pltpu.VMEM_SHARED((NS, NPAD), jnp.float32),
        pltpu.VMEM((NS, COLS), jnp.float32),
    ],
    compiler_params=_sc_params,
)(_deg_body)


def _agg_body(table, srcr, dstr, out_hbm, srcv, dstv, acc, rows, gsems, ssems):
    c = lax.axis_index("c")
    s = lax.axis_index("s")
    wid = c * NS + s
    zeros16 = jnp.zeros((16,), jnp.float32)

    def zrow(r, _):
        for k in range(D // 16):
            rows[0][r, pl.ds(k * 16, 16)] = zeros16
        return ()
    lax.fori_loop(0, CHUNK, zrow, ())
    for b in range(RPT // CHUNK):
        pltpu.sync_copy(rows[0], acc.at[pl.ds(s * RPT + b * CHUNK, CHUNK)])
    plsc.subcore_barrier()

    pltpu.sync_copy(srcr.at[wid], srcv)
    pltpu.sync_copy(dstr.at[wid], dstv)

    for b in range(LEAD):
        pltpu.async_copy(table.at[srcv.at[b]], rows[b], gsems[b])

    def outer(g, _):
        for b in range(NBUF):
            jj = g * NBUF + b
            bg = (b + LEAD) % NBUF
            pre = jj + LEAD

            @pl.when(jnp.logical_and(pre < NCHUNK, pre >= NBUF))
            def _():
                pltpu.make_async_copy(
                    rows[bg], acc.at[dstv.at[pre - NBUF]], ssems[bg]).wait()

            @pl.when(pre < NCHUNK)
            def _():
                pltpu.async_copy(table.at[srcv.at[pre]], rows[bg], gsems[bg])

            pltpu.make_async_copy(table.at[srcv.at[jj]], rows[b], gsems[b]).wait()
            pltpu.async_copy(rows[b], acc.at[dstv.at[jj]], ssems[b], add=True)
        return ()
    lax.fori_loop(0, NCHUNK // NBUF, outer, ())

    for b in range(NBUF):
        jj = NCHUNK - NBUF + b
        pltpu.make_async_copy(rows[b], acc.at[dstv.at[jj]], ssems[b]).wait()

    plsc.subcore_barrier()
    pltpu.sync_copy(acc.at[pl.ds(s * RPT, RPT)],
                    out_hbm.at[c, pl.ds(s * RPT, RPT)])


_agg = functools.partial(
    pl.kernel,
    out_type=jax.ShapeDtypeStruct((NC, NROW, D), jnp.float32),
    mesh=_mesh,
    scratch_types=[
        pltpu.VMEM((NCHUNK, CHUNK), jnp.int32),
        pltpu.VMEM((NCHUNK, CHUNK), jnp.int32),
        pltpu.VMEM_SHARED((NROW, D), jnp.float32),
        [pltpu.VMEM((CHUNK, D), jnp.float32) for _ in range(NBUF)],
        [pltpu.SemaphoreType.DMA for _ in range(NBUF)],
        [pltpu.SemaphoreType.DMA for _ in range(NBUF)],
    ],
    compiler_params=_sc_params,
)(_agg_body)


def _agg2_body(table_a, table_b, srcr, dstr, out_hbm,
               srcv, dstv, acc, rows, gsems, ssems):
    c = lax.axis_index("c")
    s = lax.axis_index("s")
    zeros16 = jnp.zeros((16,), jnp.float32)

    def zrow(r, _):
        for k in range(D // 16):
            rows[0][r, pl.ds(k * 16, 16)] = zeros16
        return ()
    lax.fori_loop(0, CHUNK, zrow, ())
    for b in range(RPT // CHUNK):
        pltpu.sync_copy(rows[0], acc.at[pl.ds(s * RPT + b * CHUNK, CHUNK)])
    plsc.subcore_barrier()

    def gissue(idx, b):
        @pl.when(c == 0)
        def _():
            pltpu.async_copy(table_a.at[idx], rows[b], gsems[b])

        @pl.when(c == 1)
        def _():
            pltpu.async_copy(table_b.at[idx], rows[b], gsems[b])

    def phase(p, _):
        pltpu.sync_copy(srcr.at[s, p], srcv)
        pltpu.sync_copy(dstr.at[s, p], dstv)

        for b in range(LEAD):
            gissue(srcv.at[b], b)

        def outer(g, _):
            for b in range(NBUF):
                jj = g * NBUF + b
                bg = (b + LEAD) % NBUF
                pre = jj + LEAD

                @pl.when(jnp.logical_and(pre < NCHUNK, pre >= NBUF))
                def _():
                    pltpu.make_async_copy(
                        rows[bg], acc.at[dstv.at[pre - NBUF]], ssems[bg]).wait()

                @pl.when(pre < NCHUNK)
                def _():
                    gissue(srcv.at[pre], bg)

                pltpu.make_async_copy(
                    table_a.at[srcv.at[jj]], rows[b], gsems[b]).wait()
                pltpu.async_copy(rows[b], acc.at[dstv.at[jj]], ssems[b], add=True)
            return ()
        lax.fori_loop(0, NCHUNK // NBUF, outer, ())

        for b in range(NBUF):
            jj = NCHUNK - NBUF + b
            pltpu.make_async_copy(rows[b], acc.at[dstv.at[jj]], ssems[b]).wait()
        return ()
    lax.fori_loop(0, 2, phase, ())

    plsc.subcore_barrier()
    pltpu.sync_copy(acc.at[pl.ds(s * RPT, RPT)],
                    out_hbm.at[c, pl.ds(s * RPT, RPT)])


_agg2 = functools.partial(
    pl.kernel,
    out_type=jax.ShapeDtypeStruct((NC, NROW, D), jnp.float32),
    mesh=_mesh,
    scratch_types=[
        pltpu.VMEM((NCHUNK, CHUNK), jnp.int32),
        pltpu.VMEM((NCHUNK, CHUNK), jnp.int32),
        pltpu.VMEM_SHARED((NROW, D), jnp.float32),
        [pltpu.VMEM((CHUNK, D), jnp.float32) for _ in range(NBUF)],
        [pltpu.SemaphoreType.DMA for _ in range(NBUF)],
        [pltpu.SemaphoreType.DMA for _ in range(NBUF)],
    ],
    compiler_params=_sc_params,
)(_agg2_body)


_BLK = 2000
_GRID = N // _BLK


def _row_spec(d):
    return pl.BlockSpec((_BLK, d), lambda i: (i, 0))


def _full_spec(a, b):
    return pl.BlockSpec((a, b), lambda i: (0, 0))


def _tc_call(body, in_specs, out_specs, out_shape):
    return pl.pallas_call(
        body, grid=(_GRID,), in_specs=in_specs,
        out_specs=out_specs, out_shape=out_shape)


def _tcA_body(c0, c1, x, w1, dinv_o, h1a_o, h1b_o):
    dinv = lax.rsqrt(c0[...] + c1[...] + 1.0)
    dinv_o[...] = dinv
    h1p = dinv * jnp.dot(x[...], w1[...], preferred_element_type=jnp.float32)
    h1a_o[...] = h1p[:, :D]
    h1b_o[...] = h1p[:, D:]


def _tcB_body(ra, rb, h1a, h1b, dinv, b1, w2, h2p_o):
    lo = dinv[...] * (ra[...] + h1a[...]) + b1[...][:, :D]
    hi = dinv[...] * (rb[...] + h1b[...]) + b1[...][:, D:]
    x2 = jnp.maximum(jnp.concatenate([lo, hi], axis=1), 0.0)
    h2p_o[...] = dinv[...] * jnp.dot(x2, w2[...], preferred_element_type=jnp.float32)


def _tcC_body(r0, r1, h2p, dinv, b2, z_o, zp_o):
    z = dinv[...] * (r0[...] + r1[...] + h2p[...]) + b2[...]
    z_o[...] = z
    zp_o[...] = dinv[...] * z


def _tcD_body(r0, r1, zp, dinv, w3, b3, w4, h4a_o, h4b_o):
    agg = dinv[...] * (r0[...] + r1[...] + zp[...])
    g = jnp.maximum(jnp.dot(agg, w3[...], preferred_element_type=jnp.float32)
                    + b3[...], 0.0)
    h4p = dinv[...] * jnp.dot(g, w4[...], preferred_element_type=jnp.float32)
    h4a_o[...] = h4p[:, :D]
    h4b_o[...] = h4p[:, D:]


def _tcE_body(ra, rb, h4a, h4b, dinv, b4, xh_o):
    lo = dinv[...] * (ra[...] + h4a[...]) + b4[...][:, :D]
    hi = dinv[...] * (rb[...] + h4b[...]) + b4[...][:, D:]
    xh_o[...] = jnp.concatenate([lo, hi], axis=1)


def kernel(x, edge_index, W1, b1, W2, b2, W3, b3, W4, b4):
    src = edge_index[0].reshape(NW, NCHUNK, CHUNK)
    dst = edge_index[1].reshape(NW, NCHUNK, CHUNK)
    src2 = edge_index[0].reshape(NS, 2, NCHUNK, CHUNK)
    dst2 = edge_index[1].reshape(NS, 2, NCHUNK, CHUNK)
    dst_flat = edge_index[1].reshape(NW, EPT)

    cnt = _deg_kernel(dst_flat)
    c0 = cnt[0].reshape(NPAD, 1)
    c1 = cnt[1].reshape(NPAD, 1)

    dinv, h1a, h1b = _tc_call(
        _tcA_body,
        [_row_spec(1), _row_spec(1), _row_spec(128), _full_spec(128, 128)],
        [_row_spec(1), _row_spec(D), _row_spec(D)],
        [jax.ShapeDtypeStruct((N, 1), jnp.float32),
         jax.ShapeDtypeStruct((N, D), jnp.float32),
         jax.ShapeDtypeStruct((N, D), jnp.float32)],
    )(c0, c1, x, W1)

    r1 = _agg2(h1a, h1b, src2, dst2)
    h2p = _tc_call(
        _tcB_body,
        [_row_spec(D)] * 2 + [_row_spec(D), _row_spec(D), _row_spec(1),
                              _full_spec(1, 128), _full_spec(128, 64)],
        [_row_spec(D)],
        [jax.ShapeDtypeStruct((N, D), jnp.float32)],
    )(r1[0], r1[1], h1a, h1b, dinv, b1.reshape(1, 128), W2)[0]

    r2 = _agg(h2p, src, dst)
    z, zp = _tc_call(
        _tcC_body,
        [_row_spec(D), _row_spec(D), _row_spec(D), _row_spec(1),
         _full_spec(1, 64)],
        [_row_spec(D), _row_spec(D)],
        [jax.ShapeDtypeStruct((N, D), jnp.float32),
         jax.ShapeDtypeStruct((N, D), jnp.float32)],
    )(r2[0], r2[1], h2p, dinv, b2.reshape(1, 64))

    r3 = _agg(zp, src, dst)
    h4a, h4b = _tc_call(
        _tcD_body,
        [_row_spec(D), _row_spec(D), _row_spec(D), _row_spec(1),
         _full_spec(64, 128), _full_spec(1, 128), _full_spec(128, 128)],
        [_row_spec(D), _row_spec(D)],
        [jax.ShapeDtypeStruct((N, D), jnp.float32),
         jax.ShapeDtypeStruct((N, D), jnp.float32)],
    )(r3[0], r3[1], zp, dinv, W3, b3.reshape(1, 128), W4)

    r4 = _agg2(h4a, h4b, src2, dst2)
    x_hat = _tc_call(
        _tcE_body,
        [_row_spec(D)] * 2 + [_row_spec(D), _row_spec(D), _row_spec(1),
                              _full_spec(1, 128)],
        [_row_spec(128)],
        [jax.ShapeDtypeStruct((N, 128), jnp.float32)],
    )(r4[0], r4[1], h4a, h4b, dinv, b4.reshape(1, 128))[0]

    return (x_hat, z)


# speedup vs baseline: 1.0841x; 1.0015x over previous
import functools

import jax
import jax.numpy as jnp
from jax import lax
from jax.experimental import pallas as pl
from jax.experimental.pallas import tpu as pltpu
from jax.experimental.pallas import tpu_sc as plsc

N = 10000
E = 320000
NPAD = 10240
NC = 2
NS = 16
NW = NC * NS
EPT = E // NW
CHUNK = 40
NCHUNK = EPT // CHUNK
NBUF = 10
LEAD = 7
NROW = 10240
RPT = NROW // NS
COLS = NPAD // NS
D = 64

_mesh = plsc.VectorSubcoreMesh(core_axis_name="c", subcore_axis_name="s")
_sc_params = pltpu.CompilerParams(
    needs_layout_passes=False, use_tc_tiling_on_sc=False,
    skip_device_barrier=True)


def _deg_body(dst_hbm, out_hbm, dstv, hist, shared, redbuf):
    c = lax.axis_index("c")
    s = lax.axis_index("s")
    wid = c * NS + s
    zeros16 = jnp.zeros((16,), jnp.float32)
    ones16 = jnp.ones((16,), jnp.float32)

    def zero(i, _):
        hist[pl.ds(i * 16, 16)] = zeros16
        return ()
    lax.fori_loop(0, NPAD // 16, zero, ())

    pltpu.sync_copy(dst_hbm.at[wid], dstv)

    def count(j, _):
        idx = dstv[pl.ds(j * 16, 16)]
        plsc.addupdate_scatter(hist, [idx], ones16)
        return ()
    lax.fori_loop(0, EPT // 16, count, ())

    pltpu.sync_copy(hist, shared.at[s])
    plsc.subcore_barrier()

    pltpu.sync_copy(shared.at[:, pl.ds(s * COLS, COLS)], redbuf)

    def red(jc, _):
        acc = jnp.zeros((16,), jnp.float32)
        for r in range(NS):
            acc = acc + redbuf[r, pl.ds(jc * 16, 16)]
        hist[pl.ds(jc * 16, 16)] = acc
        return ()
    lax.fori_loop(0, COLS // 16, red, ())
    pltpu.sync_copy(hist.at[pl.ds(0, COLS)], out_hbm.at[c, pl.ds(s * COLS, COLS)])


_deg_kernel = functools.partial(
    pl.kernel,
    out_type=jax.ShapeDtypeStruct((NC, NPAD), jnp.float32),
    mesh=_mesh,
    scratch_types=[
        pltpu.VMEM((EPT,), jnp.int32),
        pltpu.VMEM((NPAD,), jnp.float32),
        pltpu.VMEM_SHARED((NS, NPAD), jnp.float32),
        pltpu.VMEM((NS, COLS), jnp.float32),
    ],
    compiler_params=_sc_params,
)(_deg_body)


def _agg_body(table, srcr, dstr, out_hbm, srcv, dstv, acc, rows, gsems, ssems):
    c = lax.axis_index("c")
    s = lax.axis_index("s")
    wid = c * NS + s
    zeros16 = jnp.zeros((16,), jnp.float32)

    def zrow(r, _):
        for k in range(D // 16):
            rows[0][r, pl.ds(k * 16, 16)] = zeros16
        return ()
    lax.fori_loop(0, CHUNK, zrow, ())
    for b in range(RPT // CHUNK):
        pltpu.sync_copy(rows[0], acc.at[pl.ds(s * RPT + b * CHUNK, CHUNK)])
    plsc.subcore_barrier()

    pltpu.sync_copy(srcr.at[wid], srcv)
    pltpu.sync_copy(dstr.at[wid], dstv)

    for b in range(LEAD):
        pltpu.async_copy(table.at[srcv.at[b]], rows[b], gsems[b])

    def outer(g, _):
        for b in range(NBUF):
            jj = g * NBUF + b
            bg = (b + LEAD) % NBUF
            pre = jj + LEAD

            @pl.when(jnp.logical_and(pre < NCHUNK, pre >= NBUF))
            def _():
                pltpu.make_async_copy(
                    rows[bg], acc.at[dstv.at[pre - NBUF]], ssems[bg]).wait()

            @pl.when(pre < NCHUNK)
            def _():
                pltpu.async_copy(table.at[srcv.at[pre]], rows[bg], gsems[bg])

            pltpu.make_async_copy(table.at[srcv.at[jj]], rows[b], gsems[b]).wait()
            pltpu.async_copy(rows[b], acc.at[dstv.at[jj]], ssems[b], add=True)
        return ()
    lax.fori_loop(0, NCHUNK // NBUF, outer, ())

    for b in range(NBUF):
        jj = NCHUNK - NBUF + b
        pltpu.make_async_copy(rows[b], acc.at[dstv.at[jj]], ssems[b]).wait()

    plsc.subcore_barrier()
    pltpu.sync_copy(acc.at[pl.ds(s * RPT, RPT)],
                    out_hbm.at[c, pl.ds(s * RPT, RPT)])


_agg = functools.partial(
    pl.kernel,
    out_type=jax.ShapeDtypeStruct((NC, NROW, D), jnp.float32),
    mesh=_mesh,
    scratch_types=[
        pltpu.VMEM((NCHUNK, CHUNK), jnp.int32),
        pltpu.VMEM((NCHUNK, CHUNK), jnp.int32),
        pltpu.VMEM_SHARED((NROW, D), jnp.float32),
        [pltpu.VMEM((CHUNK, D), jnp.float32) for _ in range(NBUF)],
        [pltpu.SemaphoreType.DMA for _ in range(NBUF)],
        [pltpu.SemaphoreType.DMA for _ in range(NBUF)],
    ],
    compiler_params=_sc_params,
)(_agg_body)


def _agg2_body(table_a, table_b, srcr, dstr, out_hbm,
               srcv, dstv, acc, rows, gsems, ssems):
    c = lax.axis_index("c")
    s = lax.axis_index("s")
    zeros16 = jnp.zeros((16,), jnp.float32)

    def zrow(r, _):
        for k in range(D // 16):
            rows[0][r, pl.ds(k * 16, 16)] = zeros16
        return ()
    lax.fori_loop(0, CHUNK, zrow, ())
    for b in range(RPT // CHUNK):
        pltpu.sync_copy(rows[0], acc.at[pl.ds(s * RPT + b * CHUNK, CHUNK)])
    plsc.subcore_barrier()

    def gissue(idx, b):
        @pl.when(c == 0)
        def _():
            pltpu.async_copy(table_a.at[idx], rows[b], gsems[b])

        @pl.when(c == 1)
        def _():
            pltpu.async_copy(table_b.at[idx], rows[b], gsems[b])

    def phase(p, _):
        pltpu.sync_copy(srcr.at[s, p], srcv)
        pltpu.sync_copy(dstr.at[s, p], dstv)

        for b in range(LEAD):
            gissue(srcv.at[b], b)

        def outer(g, _):
            for b in range(NBUF):
                jj = g * NBUF + b
                bg = (b + LEAD) % NBUF
                pre = jj + LEAD

                @pl.when(jnp.logical_and(pre < NCHUNK, pre >= NBUF))
                def _():
                    pltpu.make_async_copy(
                        rows[bg], acc.at[dstv.at[pre - NBUF]], ssems[bg]).wait()

                @pl.when(pre < NCHUNK)
                def _():
                    gissue(srcv.at[pre], bg)

                pltpu.make_async_copy(
                    table_a.at[srcv.at[jj]], rows[b], gsems[b]).wait()
                pltpu.async_copy(rows[b], acc.at[dstv.at[jj]], ssems[b], add=True)
            return ()
        lax.fori_loop(0, NCHUNK // NBUF, outer, ())

        for b in range(NBUF):
            jj = NCHUNK - NBUF + b
            pltpu.make_async_copy(rows[b], acc.at[dstv.at[jj]], ssems[b]).wait()
        return ()
    lax.fori_loop(0, 2, phase, ())

    plsc.subcore_barrier()
    pltpu.sync_copy(acc.at[pl.ds(s * RPT, RPT)],
                    out_hbm.at[c, pl.ds(s * RPT, RPT)])


_agg2 = functools.partial(
    pl.kernel,
    out_type=jax.ShapeDtypeStruct((NC, NROW, D), jnp.float32),
    mesh=_mesh,
    scratch_types=[
        pltpu.VMEM((NCHUNK, CHUNK), jnp.int32),
        pltpu.VMEM((NCHUNK, CHUNK), jnp.int32),
        pltpu.VMEM_SHARED((NROW, D), jnp.float32),
        [pltpu.VMEM((CHUNK, D), jnp.float32) for _ in range(NBUF)],
        [pltpu.SemaphoreType.DMA for _ in range(NBUF)],
        [pltpu.SemaphoreType.DMA for _ in range(NBUF)],
    ],
    compiler_params=_sc_params,
)(_agg2_body)


_BLK = 2000
_GRID = N // _BLK


def _row_spec(d):
    return pl.BlockSpec((_BLK, d), lambda i: (i, 0))


def _full_spec(a, b):
    return pl.BlockSpec((a, b), lambda i: (0, 0))


def _tc_call(body, in_specs, out_specs, out_shape):
    return pl.pallas_call(
        body, grid=(_GRID,), in_specs=in_specs,
        out_specs=out_specs, out_shape=out_shape)


def _tcA_body(c0, c1, x, w1, dinv_o, h1a_o, h1b_o):
    dinv = lax.rsqrt(c0[...] + c1[...] + 1.0)
    dinv_o[...] = dinv
    h1p = dinv * jnp.dot(x[...], w1[...], preferred_element_type=jnp.float32)
    h1a_o[...] = h1p[:, :D]
    h1b_o[...] = h1p[:, D:]


def _tcB_body(ra, rb, h1a, h1b, dinv, b1, w2, h2p_o):
    lo = dinv[...] * (ra[...] + h1a[...]) + b1[...][:, :D]
    hi = dinv[...] * (rb[...] + h1b[...]) + b1[...][:, D:]
    x2 = jnp.maximum(jnp.concatenate([lo, hi], axis=1), 0.0)
    h2p_o[...] = dinv[...] * jnp.dot(x2, w2[...], preferred_element_type=jnp.float32)


def _tcC_body(r0, r1, h2p, dinv, b2, z_o, zp_o):
    z = dinv[...] * (r0[...] + r1[...] + h2p[...]) + b2[...]
    z_o[...] = z
    zp_o[...] = dinv[...] * z


def _tcD_body(r0, r1, zp, dinv, w3, b3, w4, h4a_o, h4b_o):
    agg = dinv[...] * (r0[...] + r1[...] + zp[...])
    g = jnp.maximum(jnp.dot(agg, w3[...], preferred_element_type=jnp.float32)
                    + b3[...], 0.0)
    h4p = dinv[...] * jnp.dot(g, w4[...], preferred_element_type=jnp.float32)
    h4a_o[...] = h4p[:, :D]
    h4b_o[...] = h4p[:, D:]


def _tcE_body(ra, rb, h4a, h4b, dinv, b4, xh_o):
    lo = dinv[...] * (ra[...] + h4a[...]) + b4[...][:, :D]
    hi = dinv[...] * (rb[...] + h4b[...]) + b4[...][:, D:]
    xh_o[...] = jnp.concatenate([lo, hi], axis=1)


def kernel(x, edge_index, W1, b1, W2, b2, W3, b3, W4, b4):
    src = edge_index[0].reshape(NW, NCHUNK, CHUNK)
    dst = edge_index[1].reshape(NW, NCHUNK, CHUNK)
    src2 = edge_index[0].reshape(NS, 2, NCHUNK, CHUNK)
    dst2 = edge_index[1].reshape(NS, 2, NCHUNK, CHUNK)
    dst_flat = edge_index[1].reshape(NW, EPT)

    cnt = _deg_kernel(dst_flat)
    c0 = cnt[0].reshape(NPAD, 1)
    c1 = cnt[1].reshape(NPAD, 1)

    dinv, h1a, h1b = _tc_call(
        _tcA_body,
        [_row_spec(1), _row_spec(1), _row_spec(128), _full_spec(128, 128)],
        [_row_spec(1), _row_spec(D), _row_spec(D)],
        [jax.ShapeDtypeStruct((N, 1), jnp.float32),
         jax.ShapeDtypeStruct((N, D), jnp.float32),
         jax.ShapeDtypeStruct((N, D), jnp.float32)],
    )(c0, c1, x, W1)

    r1 = _agg2(h1a, h1b, src2, dst2)
    h2p = _tc_call(
        _tcB_body,
        [_row_spec(D)] * 2 + [_row_spec(D), _row_spec(D), _row_spec(1),
                              _full_spec(1, 128), _full_spec(128, 64)],
        [_row_spec(D)],
        [jax.ShapeDtypeStruct((N, D), jnp.float32)],
    )(r1[0], r1[1], h1a, h1b, dinv, b1.reshape(1, 128), W2)[0]

    r2 = _agg(h2p, src, dst)
    z, zp = _tc_call(
        _tcC_body,
        [_row_spec(D), _row_spec(D), _row_spec(D), _row_spec(1),
         _full_spec(1, 64)],
        [_row_spec(D), _row_spec(D)],
        [jax.ShapeDtypeStruct((N, D), jnp.float32),
         jax.ShapeDtypeStruct((N, D), jnp.float32)],
    )(r2[0], r2[1], h2p, dinv, b2.reshape(1, 64))

    r3 = _agg(zp, src, dst)
    h4a, h4b = _tc_call(
        _tcD_body,
        [_row_spec(D), _row_spec(D), _row_spec(D), _row_spec(1),
         _full_spec(64, 128), _full_spec(1, 128), _full_spec(128, 128)],
        [_row_spec(D), _row_spec(D)],
        [jax.ShapeDtypeStruct((N, D), jnp.float32),
         jax.ShapeDtypeStruct((N, D), jnp.float32)],
    )(r3[0], r3[1], zp, dinv, W3, b3.reshape(1, 128), W4)

    r4 = _agg2(h4a, h4b, src2, dst2)
    x_hat = _tc_call(
        _tcE_body,
        [_row_spec(D)] * 2 + [_row_spec(D), _row_spec(D), _row_spec(1),
                              _full_spec(1, 128)],
        [_row_spec(128)],
        [jax.ShapeDtypeStruct((N, 128), jnp.float32)],
    )(r4[0], r4[1], h4a, h4b, dinv, b4.reshape(1, 128))[0]

    return (x_hat, z)
